# baseline (device time: 44916 ns/iter reference)
import jax
import jax.numpy as jnp
from jax import lax
from jax.experimental import pallas as pl
from jax.experimental.pallas import tpu as pltpu

M, N = 2048, 1024
AXES = ("x", "y", "z")

PARTS = (
    (0, 768, ("z", "y", "x")),
    (768, 768, ("y", "x", "z")),
    (1536, 512, ("x", "z", "y")),
)
N_PARTS = len(PARTS)
N_SEMS = 12 * N_PARTS

BF16 = jnp.bfloat16


def kernel(x):
    def body(x_ref, out_ref, rbuf, send_sems, recv_sems):
        my = {a: lax.axis_index(a) for a in AXES}

        def peer_id(axis):
            return tuple(1 - my[a] if a == axis else my[a] for a in AXES)

        barrier = pltpu.get_barrier_semaphore()
        for a in AXES:
            pl.semaphore_signal(
                barrier, inc=1,
                device_id=peer_id(a), device_id_type=pl.DeviceIdType.MESH,
            )
        pl.semaphore_wait(barrier, 3)

        ctr = [0]

        def site(src, dst, axis):
            i = ctr[0]
            ctr[0] += 1
            r = pltpu.make_async_remote_copy(
                src_ref=src,
                dst_ref=dst,
                send_sem=send_sems.at[i],
                recv_sem=recv_sems.at[i],
                device_id=peer_id(axis) if axis is not None
                else tuple(1 - my[a] for a in AXES),
                device_id_type=pl.DeviceIdType.MESH,
            )
            r.start()
            return r

        def cast(off, n):
            out_ref[pl.ds(off, n), :] = (
                x_ref[0, 0, 0, pl.ds(off, n), :].astype(BF16)
            )

        def add(off, rb_off, n):
            out_ref[pl.ds(off, n), :] = (
                out_ref[pl.ds(off, n), :] + rbuf[pl.ds(rb_off, n), :]
            )

        class S:
            pass

        parts = []
        rb = 0
        for base, rows, order in PARTS:
            s = S()
            s.order = order
            s.w = (rows // 2, rows // 4, rows // 8)
            b = [my[a] for a in order]
            s.b = b
            w0, w1, w2 = s.w
            s.K0 = base + b[0] * w0
            s.S0 = base + (1 - b[0]) * w0
            s.S0c1 = s.S0 + (1 - b[1]) * w1
            s.S0c2 = s.S0 + b[1] * w1
            s.S1 = s.K0 + (1 - b[1]) * w1
            s.K1 = s.K0 + b[1] * w1
            s.S1c1 = s.S1 + (1 - b[2]) * w2
            s.S1c2 = s.S1 + b[2] * w2
            s.S2 = s.K1 + (1 - b[2]) * w2
            s.Kf = s.K1 + b[2] * w2
            s.rb0c1, s.rb0c2 = rb, rb + w1
            s.rb1c1, s.rb1c2, s.rb2 = rb + 2 * w1, rb + 2 * w1 + w2, rb + 2 * w1 + 2 * w2
            rb += 2 * w1 + 3 * w2
            f = lambda off, k: off + (1 - 2 * b[k]) * s.w[k]
            s.f2 = f(s.Kf, 2)
            s.f1 = f(s.Kf, 1)
            s.f0 = f(s.Kf, 0)
            s.f12 = f(s.f1, 2)
            s.f02 = f(s.f0, 2)
            s.f01 = f(s.f0, 1)
            s.f012 = f(s.f01, 2)
            parts.append(s)

        for s in parts:
            w0, w1, w2 = s.w
            cast(s.S0c1, w1)
            s.r0c1 = site(
                out_ref.at[pl.ds(s.S0c1, w1)], rbuf.at[pl.ds(s.rb0c1, w1)],
                s.order[0],
            )
        for s in parts:
            w0, w1, w2 = s.w
            cast(s.S0c2, w1)
            s.r0c2 = site(
                out_ref.at[pl.ds(s.S0c2, w1)], rbuf.at[pl.ds(s.rb0c2, w1)],
                s.order[0],
            )
        for s in parts:
            cast(s.K0, s.w[0])
        for s in parts:
            w0, w1, w2 = s.w
            s.r0c1.wait()
            add(s.S1, s.rb0c1, w1)
            s.r1c1 = site(
                out_ref.at[pl.ds(s.S1c1, w2)], rbuf.at[pl.ds(s.rb1c1, w2)],
                s.order[1],
            )
            s.r1c2 = site(
                out_ref.at[pl.ds(s.S1c2, w2)], rbuf.at[pl.ds(s.rb1c2, w2)],
                s.order[1],
            )
        for s in parts:
            s.r0c2.wait()
            add(s.K1, s.rb0c2, s.w[1])
        for s in parts:
            w2 = s.w[2]
            s.r1c1.wait()
            add(s.S2, s.rb1c1, w2)
            s.r2 = site(
                out_ref.at[pl.ds(s.S2, w2)], rbuf.at[pl.ds(s.rb2, w2)],
                s.order[2],
            )
        for s in parts:
            s.r1c2.wait()
            add(s.Kf, s.rb1c2, s.w[2])
        for s in parts:
            s.r2.wait()
            add(s.Kf, s.rb2, s.w[2])

        def ag(src_off, axis, w2):
            return site(
                out_ref.at[pl.ds(src_off, w2)], out_ref.at[pl.ds(src_off, w2)],
                axis,
            )

        for s in parts:
            w2 = s.w[2]
            s.a0 = ag(s.Kf, s.order[2], w2)
            s.a1 = ag(s.Kf, s.order[1], w2)
            s.a2 = ag(s.Kf, s.order[0], w2)
            s.a6 = ag(s.Kf, None, w2)
        for s in parts:
            w2 = s.w[2]
            s.a0.wait()
            s.a3 = ag(s.f2, s.order[1], w2)
            s.a4 = ag(s.f2, s.order[0], w2)
        for s in parts:
            s.a1.wait()
            s.a5 = ag(s.f1, s.order[0], s.w[2])
        for s in parts:
            s.a2.wait()
            s.a3.wait()
            s.a4.wait()
            s.a5.wait()
            s.a6.wait()

    return pl.pallas_call(
        body,
        out_shape=jax.ShapeDtypeStruct((M, N), BF16),
        in_specs=[pl.BlockSpec(memory_space=pltpu.VMEM)],
        out_specs=pl.BlockSpec(memory_space=pltpu.VMEM),
        scratch_shapes=[
            pltpu.VMEM((1792, N), BF16),
            pltpu.SemaphoreType.DMA((N_SEMS,)),
            pltpu.SemaphoreType.DMA((N_SEMS,)),
        ],
        compiler_params=pltpu.CompilerParams(collective_id=0),
    )(x)


# device time: 41576 ns/iter; 1.0803x vs baseline; 1.0803x over previous
import jax
import jax.numpy as jnp
from jax import lax
from jax.experimental import pallas as pl
from jax.experimental.pallas import tpu as pltpu

M, N = 2048, 1024
AXES = ("x", "y", "z")

PARTS = (
    (0, 768, ("z", "y", "x")),
    (768, 768, ("y", "x", "z")),
    (1536, 512, ("x", "z", "y")),
)
N_PARTS = len(PARTS)
N_SEMS = 12 * N_PARTS

BF16 = jnp.bfloat16


def kernel(x):
    def body(x_ref, out_ref, rbuf, send_sems, recv_sems):
        my = {a: lax.axis_index(a) for a in AXES}

        def peer_id(axis):
            return tuple(1 - my[a] if a == axis else my[a] for a in AXES)

        barrier = pltpu.get_barrier_semaphore()
        for a in AXES:
            pl.semaphore_signal(
                barrier, inc=1,
                device_id=peer_id(a), device_id_type=pl.DeviceIdType.MESH,
            )
        pl.semaphore_wait(barrier, 3)

        ctr = [0]

        def site(src, dst, axis):
            i = ctr[0]
            ctr[0] += 1
            r = pltpu.make_async_remote_copy(
                src_ref=src,
                dst_ref=dst,
                send_sem=send_sems.at[i],
                recv_sem=recv_sems.at[i],
                device_id=peer_id(axis),
                device_id_type=pl.DeviceIdType.MESH,
            )
            r.start()
            return r

        def cast(off, n):
            out_ref[pl.ds(off, n), :] = (
                x_ref[0, 0, 0, pl.ds(off, n), :].astype(BF16)
            )

        def add(off, rb_off, n):
            out_ref[pl.ds(off, n), :] = (
                out_ref[pl.ds(off, n), :] + rbuf[pl.ds(rb_off, n), :]
            )

        class S:
            pass

        parts = []
        rb = 0
        for base, rows, order in PARTS:
            s = S()
            s.order = order
            s.w = (rows // 2, rows // 4, rows // 8)
            b = [my[a] for a in order]
            s.b = b
            w0, w1, w2 = s.w
            s.K0 = base + b[0] * w0
            s.S0 = base + (1 - b[0]) * w0
            s.S0c1 = s.S0 + (1 - b[1]) * w1
            s.S0c2 = s.S0 + b[1] * w1
            s.S1 = s.K0 + (1 - b[1]) * w1
            s.K1 = s.K0 + b[1] * w1
            s.S1c1 = s.S1 + (1 - b[2]) * w2
            s.S1c2 = s.S1 + b[2] * w2
            s.S2 = s.K1 + (1 - b[2]) * w2
            s.Kf = s.K1 + b[2] * w2
            s.rb0c1, s.rb0c2 = rb, rb + w1
            s.rb1c1, s.rb1c2, s.rb2 = rb + 2 * w1, rb + 2 * w1 + w2, rb + 2 * w1 + 2 * w2
            rb += 2 * w1 + 3 * w2
            f = lambda off, k: off + (1 - 2 * b[k]) * s.w[k]
            s.f2 = f(s.Kf, 2)
            s.f1 = f(s.Kf, 1)
            s.f0 = f(s.Kf, 0)
            s.f12 = f(s.f1, 2)
            s.f02 = f(s.f0, 2)
            s.f01 = f(s.f0, 1)
            s.f012 = f(s.f01, 2)
            parts.append(s)

        for s in parts:
            w0, w1, w2 = s.w
            cast(s.S0c1, w1)
            s.r0c1 = site(
                out_ref.at[pl.ds(s.S0c1, w1)], rbuf.at[pl.ds(s.rb0c1, w1)],
                s.order[0],
            )
        for s in parts:
            w0, w1, w2 = s.w
            cast(s.S0c2, w1)
            s.r0c2 = site(
                out_ref.at[pl.ds(s.S0c2, w1)], rbuf.at[pl.ds(s.rb0c2, w1)],
                s.order[0],
            )
        for s in parts:
            cast(s.K0, s.w[0])
        for s in parts:
            w0, w1, w2 = s.w
            s.r0c1.wait()
            add(s.S1, s.rb0c1, w1)
            s.r1c1 = site(
                out_ref.at[pl.ds(s.S1c1, w2)], rbuf.at[pl.ds(s.rb1c1, w2)],
                s.order[1],
            )
            s.r1c2 = site(
                out_ref.at[pl.ds(s.S1c2, w2)], rbuf.at[pl.ds(s.rb1c2, w2)],
                s.order[1],
            )
        for s in parts:
            s.r0c2.wait()
            add(s.K1, s.rb0c2, s.w[1])
        for s in parts:
            w2 = s.w[2]
            s.r1c1.wait()
            add(s.S2, s.rb1c1, w2)
            s.r2 = site(
                out_ref.at[pl.ds(s.S2, w2)], rbuf.at[pl.ds(s.rb2, w2)],
                s.order[2],
            )
        for s in parts:
            s.r1c2.wait()
            add(s.Kf, s.rb1c2, s.w[2])
        for s in parts:
            s.r2.wait()
            add(s.Kf, s.rb2, s.w[2])

        def ag(src_off, axis, w2):
            return site(
                out_ref.at[pl.ds(src_off, w2)], out_ref.at[pl.ds(src_off, w2)],
                axis,
            )

        for s in parts:
            w2 = s.w[2]
            s.a0 = ag(s.Kf, s.order[0], w2)
            s.a1 = ag(s.Kf, s.order[1], w2)
            s.a2 = ag(s.Kf, s.order[2], w2)
        for s in parts:
            w2 = s.w[2]
            s.a0.wait()
            s.a3 = ag(s.f0, s.order[1], w2)
            s.a4 = ag(s.f0, s.order[2], w2)
        for s in parts:
            s.a1.wait()
            s.a5 = ag(s.f1, s.order[2], s.w[2])
        for s in parts:
            s.a3.wait()
            s.a6 = ag(s.f01, s.order[2], s.w[2])
        for s in parts:
            s.a2.wait()
            s.a4.wait()
            s.a5.wait()
            s.a6.wait()

    return pl.pallas_call(
        body,
        out_shape=jax.ShapeDtypeStruct((M, N), BF16),
        in_specs=[pl.BlockSpec(memory_space=pltpu.VMEM)],
        out_specs=pl.BlockSpec(memory_space=pltpu.VMEM),
        scratch_shapes=[
            pltpu.VMEM((1792, N), BF16),
            pltpu.SemaphoreType.DMA((N_SEMS,)),
            pltpu.SemaphoreType.DMA((N_SEMS,)),
        ],
        compiler_params=pltpu.CompilerParams(collective_id=0),
    )(x)


# device time: 41143 ns/iter; 1.0917x vs baseline; 1.0105x over previous
import jax
import jax.numpy as jnp
from jax import lax
from jax.experimental import pallas as pl
from jax.experimental.pallas import tpu as pltpu

M, N = 2048, 1024
AXES = ("x", "y", "z")

PARTS = (
    (0, 768, ("z", "y", "x")),
    (768, 768, ("y", "x", "z")),
    (1536, 512, ("x", "z", "y")),
)
N_PARTS = len(PARTS)
N_SEMS = 13 * N_PARTS

BF16 = jnp.bfloat16


def kernel(x):
    def body(x_ref, out_ref, rbuf, send_sems, recv_sems):
        my = {a: lax.axis_index(a) for a in AXES}

        def peer_id(axis):
            return tuple(1 - my[a] if a == axis else my[a] for a in AXES)

        barrier = pltpu.get_barrier_semaphore()
        for a in AXES:
            pl.semaphore_signal(
                barrier, inc=1,
                device_id=peer_id(a), device_id_type=pl.DeviceIdType.MESH,
            )
        pl.semaphore_wait(barrier, 3)

        ctr = [0]

        def site(src, dst, axis):
            i = ctr[0]
            ctr[0] += 1
            r = pltpu.make_async_remote_copy(
                src_ref=src,
                dst_ref=dst,
                send_sem=send_sems.at[i],
                recv_sem=recv_sems.at[i],
                device_id=peer_id(axis),
                device_id_type=pl.DeviceIdType.MESH,
            )
            r.start()
            return r

        def cast(off, n):
            out_ref[pl.ds(off, n), :] = (
                x_ref[0, 0, 0, pl.ds(off, n), :].astype(BF16)
            )

        def add(off, rb_off, n):
            out_ref[pl.ds(off, n), :] = (
                out_ref[pl.ds(off, n), :] + rbuf[pl.ds(rb_off, n), :]
            )

        class S:
            pass

        parts = []
        rb = 0
        for base, rows, order in PARTS:
            s = S()
            s.order = order
            s.w = (rows // 2, rows // 4, rows // 8)
            b = [my[a] for a in order]
            s.b = b
            w0, w1, w2 = s.w
            s.K0 = base + b[0] * w0
            s.S0 = base + (1 - b[0]) * w0
            s.S0c1 = s.S0 + (1 - b[1]) * w1
            s.S0c2 = s.S0 + b[1] * w1
            s.S1 = s.K0 + (1 - b[1]) * w1
            s.K1 = s.K0 + b[1] * w1
            s.S1c1 = s.S1 + (1 - b[2]) * w2
            s.S1c2 = s.S1 + b[2] * w2
            s.S2 = s.K1 + (1 - b[2]) * w2
            s.Kf = s.K1 + b[2] * w2
            s.S0c1a = s.S0c1 + (1 - b[2]) * w2
            s.S0c1b = s.S0c1 + b[2] * w2
            s.rb0c1a, s.rb0c1b, s.rb0c2 = rb, rb + w2, rb + 2 * w2
            s.rb1c1 = rb + 2 * w2 + w1
            s.rb1c2 = s.rb1c1 + w2
            s.rb2 = s.rb1c2 + w2
            rb += w1 + 5 * w2
            f = lambda off, k: off + (1 - 2 * b[k]) * s.w[k]
            s.f2 = f(s.Kf, 2)
            s.f1 = f(s.Kf, 1)
            s.f0 = f(s.Kf, 0)
            s.f12 = f(s.f1, 2)
            s.f02 = f(s.f0, 2)
            s.f01 = f(s.f0, 1)
            s.f012 = f(s.f01, 2)
            parts.append(s)

        for s in parts:
            w2 = s.w[2]
            cast(s.S0c1a, w2)
            s.r0c1a = site(
                out_ref.at[pl.ds(s.S0c1a, w2)], rbuf.at[pl.ds(s.rb0c1a, w2)],
                s.order[0],
            )
        for s in parts:
            w2 = s.w[2]
            cast(s.S0c1b, w2)
            s.r0c1b = site(
                out_ref.at[pl.ds(s.S0c1b, w2)], rbuf.at[pl.ds(s.rb0c1b, w2)],
                s.order[0],
            )
        for s in parts:
            w1 = s.w[1]
            cast(s.S0c2, w1)
            s.r0c2 = site(
                out_ref.at[pl.ds(s.S0c2, w1)], rbuf.at[pl.ds(s.rb0c2, w1)],
                s.order[0],
            )
        for s in parts:
            cast(s.K0, s.w[0])
        for s in parts:
            w2 = s.w[2]
            s.r0c1a.wait()
            add(s.S1c1, s.rb0c1a, w2)
            s.r1c1 = site(
                out_ref.at[pl.ds(s.S1c1, w2)], rbuf.at[pl.ds(s.rb1c1, w2)],
                s.order[1],
            )
        for s in parts:
            w2 = s.w[2]
            s.r0c1b.wait()
            add(s.S1c2, s.rb0c1b, w2)
            s.r1c2 = site(
                out_ref.at[pl.ds(s.S1c2, w2)], rbuf.at[pl.ds(s.rb1c2, w2)],
                s.order[1],
            )
        for s in parts:
            s.r0c2.wait()
            add(s.K1, s.rb0c2, s.w[1])
        for s in parts:
            w2 = s.w[2]
            s.r1c1.wait()
            add(s.S2, s.rb1c1, w2)
            s.r2 = site(
                out_ref.at[pl.ds(s.S2, w2)], rbuf.at[pl.ds(s.rb2, w2)],
                s.order[2],
            )
        for s in parts:
            s.r1c2.wait()
            add(s.Kf, s.rb1c2, s.w[2])
        for s in parts:
            s.r2.wait()
            add(s.Kf, s.rb2, s.w[2])

        def ag(src_off, axis, w2):
            return site(
                out_ref.at[pl.ds(src_off, w2)], out_ref.at[pl.ds(src_off, w2)],
                axis,
            )

        for s in parts:
            w2 = s.w[2]
            s.a0 = ag(s.Kf, s.order[0], w2)
            s.a1 = ag(s.Kf, s.order[1], w2)
            s.a2 = ag(s.Kf, s.order[2], w2)
        for s in parts:
            w2 = s.w[2]
            s.a0.wait()
            s.a3 = ag(s.f0, s.order[1], w2)
            s.a4 = ag(s.f0, s.order[2], w2)
        for s in parts:
            s.a1.wait()
            s.a5 = ag(s.f1, s.order[2], s.w[2])
        for s in parts:
            s.a3.wait()
            s.a6 = ag(s.f01, s.order[2], s.w[2])
        for s in parts:
            s.a2.wait()
            s.a4.wait()
            s.a5.wait()
            s.a6.wait()

    return pl.pallas_call(
        body,
        out_shape=jax.ShapeDtypeStruct((M, N), BF16),
        in_specs=[pl.BlockSpec(memory_space=pltpu.VMEM)],
        out_specs=pl.BlockSpec(memory_space=pltpu.VMEM),
        scratch_shapes=[
            pltpu.VMEM((1792, N), BF16),
            pltpu.SemaphoreType.DMA((N_SEMS,)),
            pltpu.SemaphoreType.DMA((N_SEMS,)),
        ],
        compiler_params=pltpu.CompilerParams(collective_id=0),
    )(x)
